# baseline (speedup 1.0000x reference)
import functools
import jax
import jax.numpy as jnp
from jax.experimental import pallas as pl
from jax.experimental.pallas import tpu as pltpu

H = 224
W = 224
HO = H // 2
WO = W // 2


def _body(xt_ref, out_ref, s_ref):
    s_ref[...] = jnp.zeros((WO, H), jnp.float32)
    k_iota = jax.lax.broadcasted_iota(jnp.int32, (W, 1), 0)
    w_iota = jax.lax.broadcasted_iota(jnp.int32, (WO, 1), 0)

    def step(i, _):
        xt = xt_ref[0]
        rowi = xt_ref[0, pl.ds(i, 1), :]
        fi = i.astype(jnp.float32)
        lt = (xt < rowi).astype(jnp.float32)
        tie = jnp.where(k_iota < i, (xt == rowi).astype(jnp.float32), 0.0)
        r = jnp.sum(lt + tie, axis=0, keepdims=True)
        b = r.astype(jnp.int32) // 2
        s_ref[...] += fi * (b == w_iota).astype(jnp.float32)
        return 0

    jax.lax.fori_loop(0, W, step, 0)
    h_pair = jnp.where(
        jax.lax.broadcasted_iota(jnp.int32, (H, HO), 0) // 2
        == jax.lax.broadcasted_iota(jnp.int32, (H, HO), 1),
        jnp.float32(1.0),
        jnp.float32(0.0),
    )
    out_ref[0] = jnp.dot(s_ref[...], h_pair, preferred_element_type=jnp.float32) * 0.25


@jax.jit
def kernel(x):
    b, c, h, w = x.shape
    imgs = b * c
    xt = jnp.swapaxes(x.reshape(imgs, h, w), 1, 2)
    out_t = pl.pallas_call(
        _body,
        grid=(imgs,),
        in_specs=[pl.BlockSpec((1, W, H), lambda i: (i, 0, 0))],
        out_specs=pl.BlockSpec((1, WO, HO), lambda i: (i, 0, 0)),
        out_shape=jax.ShapeDtypeStruct((imgs, WO, HO), jnp.float32),
        scratch_shapes=[pltpu.VMEM((WO, H), jnp.float32)],
    )(xt)
    return jnp.swapaxes(out_t, 1, 2).reshape(b, c, HO, WO)


# speedup vs baseline: 1.4168x; 1.4168x over previous
import functools
import jax
import jax.numpy as jnp
from jax import lax
from jax.experimental import pallas as pl
from jax.experimental.pallas import tpu as pltpu, tpu_sc as plsc

H = 224
W = 224
HO = H // 2
WO = W // 2
NV = W // 16
NB = 256
NBV = NB // 16
IMGS = 768
IMGS_PER_WORKER = IMGS // 32

_MESH = plsc.VectorSubcoreMesh(core_axis_name="c", subcore_axis_name="s")


def _sortable_i32(xf):
    xb = plsc.bitcast(xf, jnp.int32)
    flip = (xb >> 31) | jnp.int32(-(2**31))
    return xb ^ flip


def _digit(key_i, shift):
    d = (plsc.bitcast(key_i, jnp.uint32) >> jnp.uint32(shift)) & jnp.uint32(NB - 1)
    return plsc.bitcast(d, jnp.int32)


def _body(x_hbm, out_hbm, ximg, oimg, ka0, kb0, va0, vb0, hist0, base0,
          ka1, kb1, va1, vb1, hist1, base1):
    wid = lax.axis_index("s") * 2 + lax.axis_index("c")
    iota = lax.iota(jnp.int32, 16)

    def sort_row(h, kA, kB, vA, vB, hist, base):
        for i in range(NBV):
            hist[pl.ds(16 * i, 16)] = jnp.zeros((16,), jnp.int32)
        for v in range(NV):
            key = _sortable_i32(ximg[h, pl.ds(16 * v, 16)])
            kA[pl.ds(16 * v, 16)] = key
            d = _digit(key, 0)
            rc, last = plsc.scan_count(d)
            plsc.addupdate_scatter(hist, [d], rc, mask=last)
        carry = jnp.int32(0)
        for i in range(NBV):
            hv = hist[pl.ds(16 * i, 16)]
            c = plsc.cumsum(hv)
            base[pl.ds(16 * i, 16)] = c - hv + carry
            carry = carry + jnp.sum(hv)
        for v in range(NV):
            k = kA[pl.ds(16 * v, 16)]
            d = _digit(k, 0)
            rc, last = plsc.scan_count(d)
            pos = plsc.load_gather(base, [d]) + rc - 1
            plsc.store_scatter(kB, [pos], k)
            plsc.store_scatter(vB, [pos], iota + jnp.int32(16 * v))
            plsc.addupdate_scatter(base, [d], rc, mask=last)
        for p, (sk, sv, dk, dv) in enumerate(
            [(kB, vB, kA, vA), (kA, vA, kB, vB), (kB, vB, kA, vA)], start=1
        ):
            shift = 8 * p
            for i in range(NBV):
                hist[pl.ds(16 * i, 16)] = jnp.zeros((16,), jnp.int32)
            for v in range(NV):
                d = _digit(sk[pl.ds(16 * v, 16)], shift)
                rc, last = plsc.scan_count(d)
                plsc.addupdate_scatter(hist, [d], rc, mask=last)
            carry = jnp.int32(0)
            for i in range(NBV):
                hv = hist[pl.ds(16 * i, 16)]
                c = plsc.cumsum(hv)
                base[pl.ds(16 * i, 16)] = c - hv + carry
                carry = carry + jnp.sum(hv)
            for v in range(NV):
                k = sk[pl.ds(16 * v, 16)]
                val = sv[pl.ds(16 * v, 16)]
                d = _digit(k, shift)
                rc, last = plsc.scan_count(d)
                pos = plsc.load_gather(base, [d]) + rc - 1
                if p < 3:
                    plsc.store_scatter(dk, [pos], k)
                plsc.store_scatter(dv, [pos], val)
                plsc.addupdate_scatter(base, [d], rc, mask=last)
        s = []
        for m in range(WO // 16):
            idx = iota * 2 + jnp.int32(32 * m)
            e = plsc.load_gather(vA, [idx])
            o = plsc.load_gather(vA, [idx + 1])
            s.append(e + o)
        return s

    def img_body(jj, _):
        img = wid * IMGS_PER_WORKER + jj
        pltpu.sync_copy(x_hbm.at[pl.ds(img * H, H)], ximg)

        def pair_body(hp, _):
            s0 = sort_row(2 * hp, ka0, kb0, va0, vb0, hist0, base0)
            s1 = sort_row(2 * hp + 1, ka1, kb1, va1, vb1, hist1, base1)
            for m in range(WO // 16):
                tot = (s0[m] + s1[m]).astype(jnp.float32) * 0.25
                oimg[pl.ds(hp * WO + 16 * m, 16)] = tot
            return 0

        lax.fori_loop(0, HO, pair_body, 0)
        pltpu.sync_copy(oimg, out_hbm.at[img])
        return 0

    lax.fori_loop(0, IMGS_PER_WORKER, img_body, 0)


@functools.partial(
    pl.kernel,
    out_type=jax.ShapeDtypeStruct((IMGS, HO * WO), jnp.float32),
    mesh=_MESH,
    compiler_params=pltpu.CompilerParams(needs_layout_passes=False),
    scratch_types=[
        pltpu.VMEM((H, W), jnp.float32),
        pltpu.VMEM((HO * WO,), jnp.float32),
    ] + 2 * [
        pltpu.VMEM((W,), jnp.int32),
        pltpu.VMEM((W,), jnp.int32),
        pltpu.VMEM((W,), jnp.int32),
        pltpu.VMEM((W,), jnp.int32),
        pltpu.VMEM((NB,), jnp.int32),
        pltpu.VMEM((NB,), jnp.int32),
    ],
)
def _sc_kernel(x_hbm, out_hbm, *scratch):
    _body(x_hbm, out_hbm, *scratch)


@jax.jit
def kernel(x):
    b, c, h, w = x.shape
    xf = x.reshape(b * c * h, w)
    out = _sc_kernel(xf)
    return out.reshape(b, c, HO, WO)


# speedup vs baseline: 1.6503x; 1.1648x over previous
import functools
import jax
import jax.numpy as jnp
from jax import lax
from jax.experimental import pallas as pl
from jax.experimental.pallas import tpu as pltpu, tpu_sc as plsc

H = 224
W = 224
HO = H // 2
WO = W // 2
NV = W // 16
NB = 256
NBV = NB // 16
IMGS = 768
IMGS_PER_WORKER = IMGS // 32

_MESH = plsc.VectorSubcoreMesh(core_axis_name="c", subcore_axis_name="s")


def _sortable_i32(xf):
    xb = plsc.bitcast(xf, jnp.int32)
    flip = (xb >> 31) | jnp.int32(-(2**31))
    return xb ^ flip


def _digit(key_i, shift):
    d = (plsc.bitcast(key_i, jnp.uint32) >> jnp.uint32(shift)) & jnp.uint32(NB - 1)
    return plsc.bitcast(d, jnp.int32)


def _body(x_hbm, out_hbm, ximg, oimg, ka0, kb0, va0, vb0, hist0, base0,
          ka1, kb1, va1, vb1, hist1, base1):
    wid = lax.axis_index("s") * 2 + lax.axis_index("c")
    iota = lax.iota(jnp.int32, 16)

    ones = jnp.ones((16,), jnp.int32)

    def sort_row(h, kA, kB, vA, vB, hist, base):
        for i in range(NBV):
            hist[pl.ds(16 * i, 16)] = jnp.zeros((16,), jnp.int32)
        for v in range(NV):
            key = _sortable_i32(ximg[h, pl.ds(16 * v, 16)])
            kA[pl.ds(16 * v, 16)] = key
            plsc.addupdate_scatter(hist, [_digit(key, 0)], ones)
        carry = jnp.int32(0)
        for i in range(NBV):
            hv = hist[pl.ds(16 * i, 16)]
            c = plsc.cumsum(hv)
            base[pl.ds(16 * i, 16)] = c - hv + carry
            carry = carry + c[15]
        for v in range(NV):
            k = kA[pl.ds(16 * v, 16)]
            d = _digit(k, 0)
            rc, last = plsc.scan_count(d)
            pos = plsc.load_gather(base, [d]) + rc - 1
            plsc.store_scatter(kB, [pos], k)
            plsc.store_scatter(vB, [pos], iota + jnp.int32(16 * v))
            plsc.addupdate_scatter(base, [d], rc, mask=last)
        for p, (sk, sv, dk, dv) in enumerate(
            [(kB, vB, kA, vA), (kA, vA, kB, vB), (kB, vB, kA, vA)], start=1
        ):
            shift = 8 * p
            for i in range(NBV):
                hist[pl.ds(16 * i, 16)] = jnp.zeros((16,), jnp.int32)
            for v in range(NV):
                plsc.addupdate_scatter(
                    hist, [_digit(sk[pl.ds(16 * v, 16)], shift)], ones)
            carry = jnp.int32(0)
            for i in range(NBV):
                hv = hist[pl.ds(16 * i, 16)]
                c = plsc.cumsum(hv)
                base[pl.ds(16 * i, 16)] = c - hv + carry
                carry = carry + c[15]
            for v in range(NV):
                k = sk[pl.ds(16 * v, 16)]
                val = sv[pl.ds(16 * v, 16)]
                d = _digit(k, shift)
                rc, last = plsc.scan_count(d)
                pos = plsc.load_gather(base, [d]) + rc - 1
                if p < 3:
                    plsc.store_scatter(dk, [pos], k)
                plsc.store_scatter(dv, [pos], val)
                plsc.addupdate_scatter(base, [d], rc, mask=last)
        s = []
        for m in range(WO // 16):
            idx = iota * 2 + jnp.int32(32 * m)
            e = plsc.load_gather(vA, [idx])
            o = plsc.load_gather(vA, [idx + 1])
            s.append(e + o)
        return s

    def img_body(jj, _):
        img = wid * IMGS_PER_WORKER + jj
        pltpu.sync_copy(x_hbm.at[pl.ds(img * H, H)], ximg)

        def pair_body(hp, _):
            s0 = sort_row(2 * hp, ka0, kb0, va0, vb0, hist0, base0)
            s1 = sort_row(2 * hp + 1, ka1, kb1, va1, vb1, hist1, base1)
            for m in range(WO // 16):
                tot = (s0[m] + s1[m]).astype(jnp.float32) * 0.25
                oimg[pl.ds(hp * WO + 16 * m, 16)] = tot
            return 0

        lax.fori_loop(0, HO, pair_body, 0)
        pltpu.sync_copy(oimg, out_hbm.at[img])
        return 0

    lax.fori_loop(0, IMGS_PER_WORKER, img_body, 0)


@functools.partial(
    pl.kernel,
    out_type=jax.ShapeDtypeStruct((IMGS, HO * WO), jnp.float32),
    mesh=_MESH,
    compiler_params=pltpu.CompilerParams(needs_layout_passes=False),
    scratch_types=[
        pltpu.VMEM((H, W), jnp.float32),
        pltpu.VMEM((HO * WO,), jnp.float32),
    ] + 2 * [
        pltpu.VMEM((W,), jnp.int32),
        pltpu.VMEM((W,), jnp.int32),
        pltpu.VMEM((W,), jnp.int32),
        pltpu.VMEM((W,), jnp.int32),
        pltpu.VMEM((NB,), jnp.int32),
        pltpu.VMEM((NB,), jnp.int32),
    ],
)
def _sc_kernel(x_hbm, out_hbm, *scratch):
    _body(x_hbm, out_hbm, *scratch)


@jax.jit
def kernel(x):
    b, c, h, w = x.shape
    xf = x.reshape(b * c * h, w)
    out = _sc_kernel(xf)
    return out.reshape(b, c, HO, WO)
